# baseline (speedup 1.0000x reference)
import functools

import jax
import jax.numpy as jnp
from jax import lax
from jax.experimental import pallas as pl
from jax.experimental.pallas import tpu as pltpu
from jax.experimental.pallas import tpu_sc as plsc

EMBED_DIM = 128
NC = 2
NS = 16
NW = NC * NS
NB_ROWS = 4
NBUF = 4


@functools.lru_cache(maxsize=None)
def _make_kernel(B: int, H: int):
    assert B % (NW * NB_ROWS) == 0 and H <= 128
    rows_per_worker = B // NW
    n_slots = rows_per_worker // NB_ROWS
    assert n_slots > NBUF
    mesh = plsc.VectorSubcoreMesh(core_axis_name="c", subcore_axis_name="s")

    @functools.partial(
        pl.kernel,
        mesh=mesh,
        out_type=jax.ShapeDtypeStruct((B, H, EMBED_DIM), jnp.float32),
        scratch_types=[
            pltpu.VMEM((rows_per_worker, H), jnp.int32),
            pltpu.VMEM((NBUF, NB_ROWS, H, EMBED_DIM), jnp.float32),
            pltpu.SemaphoreType.DMA((NBUF,)),
            pltpu.SemaphoreType.DMA((NBUF,)),
        ],
        compiler_params=pltpu.CompilerParams(use_tc_tiling_on_sc=True),
    )
    def k(idx_hbm, table_hbm, out_hbm, idx_v, rows_v, gsem, ssem):
        wid = lax.axis_index("s") * NC + lax.axis_index("c")
        pltpu.sync_copy(idx_hbm.at[wid], idx_v)
        row0 = wid * rows_per_worker

        def fire_gathers(c):
            b = c % NBUF
            for s in range(NB_ROWS):
                pltpu.async_copy(
                    table_hbm.at[idx_v.at[c * NB_ROWS + s]],
                    rows_v.at[b, s],
                    gsem.at[b],
                )

        def drain_gathers(c):
            b = c % NBUF
            for s in range(NB_ROWS):
                pltpu.make_async_copy(
                    table_hbm.at[idx_v.at[c * NB_ROWS + s]],
                    rows_v.at[b, s],
                    gsem.at[b],
                ).wait()

        def fire_scatter(c):
            b = c % NBUF
            pltpu.async_copy(
                rows_v.at[b],
                out_hbm.at[pl.ds(row0 + c * NB_ROWS, NB_ROWS)],
                ssem.at[b],
            )

        def drain_scatter(c):
            b = c % NBUF
            pltpu.make_async_copy(
                rows_v.at[b],
                out_hbm.at[pl.ds(row0 + c * NB_ROWS, NB_ROWS)],
                ssem.at[b],
            ).wait()

        fire_gathers(0)
        fire_gathers(1)

        def body(c, carry):
            drain_gathers(c)
            fire_scatter(c)
            @pl.when(c >= NBUF - 2)
            def _():
                drain_scatter(c - (NBUF - 2))

            @pl.when(c + 2 < n_slots)
            def _():
                fire_gathers(c + 2)

            return carry

        lax.fori_loop(0, n_slots, body, 0)
        for t in range(NBUF - 2):
            drain_scatter(n_slots - (NBUF - 2) + t)

    return k


@jax.jit
def kernel(indices, species_table):
    B, H = indices.shape
    idx3d = indices.reshape(NW, B // NW, H).astype(jnp.int32)
    return _make_kernel(B, H)(idx3d, species_table)
